# baseline (speedup 1.0000x reference)
import functools

import jax
import jax.numpy as jnp
from jax import lax
from jax.experimental import pallas as pl
from jax.experimental.pallas import tpu as pltpu
from jax.experimental.pallas import tpu_sc as plsc

N = 10000
E = 320000
D_IN = 128
D_H = 128
N_CLASSES = 40
C_PAD = 128
N_PAD = 10240
NC, NS = 2, 16
NW = NC * NS
CHUNK = 128
NCHUNKS = 2560
E_PAD = NCHUNKS * CHUNK
CPW = NCHUNKS // NW
RPS = N_PAD // NS
SBLK = 40
CPW0, CPW1 = 40, 120
BLK = 256

_mesh = plsc.VectorSubcoreMesh(core_axis_name="c", subcore_axis_name="s")


def _make_seg(d):
    @functools.partial(
        pl.kernel,
        out_type=jax.ShapeDtypeStruct((NC * N_PAD, d), jnp.float32),
        mesh=_mesh,
        scratch_types=[
            pltpu.VMEM((SBLK, CHUNK), jnp.int32),
            pltpu.VMEM((SBLK, CHUNK), jnp.int32),
            pltpu.VMEM((CHUNK, d), jnp.float32),
            pltpu.VMEM((CHUNK, d), jnp.float32),
            pltpu.VMEM_SHARED((N_PAD, d), jnp.float32),
            pltpu.SemaphoreType.DMA,
            pltpu.SemaphoreType.DMA,
        ],
    )
    def seg(t_hbm, src_hbm, dst_hbm, zero_hbm, out_hbm, src_v, dst_v,
            rows0, rows1, acc_sh, ss0, ss1):
        c = lax.axis_index("c")
        s = lax.axis_index("s")
        pltpu.sync_copy(zero_hbm, acc_sh.at[pl.ds(s * RPS, RPS)])
        plsc.subcore_barrier()

        bufs = (rows0, rows1)
        sems = (ss0, ss1)

        def pipeline(cpw, base_w):
            for h in range(cpw // SBLK):
                base = base_w + h * SBLK
                pltpu.sync_copy(src_hbm.at[pl.ds(base, SBLK)], src_v)
                pltpu.sync_copy(dst_hbm.at[pl.ds(base, SBLK)], dst_v)
                for p in (0, 1):
                    pltpu.sync_copy(t_hbm.at[src_v.at[p]], bufs[p])
                    pltpu.async_copy(bufs[p], acc_sh.at[dst_v.at[p]],
                                     sems[p], add=True)

                @pl.loop(2, SBLK, step=2)
                def _(j):
                    for p in (0, 1):
                        jj = j + p
                        pltpu.make_async_copy(t_hbm.at[src_v.at[jj]], bufs[p],
                                              sems[p]).wait()
                        pltpu.sync_copy(t_hbm.at[src_v.at[jj]], bufs[p])
                        pltpu.async_copy(bufs[p], acc_sh.at[dst_v.at[jj]],
                                         sems[p], add=True)

                for p in (0, 1):
                    pltpu.make_async_copy(t_hbm.at[src_v.at[p]], bufs[p],
                                          sems[p]).wait()

        @pl.when(c == 0)
        def _():
            pipeline(CPW0, s * CPW0)

        @pl.when(c == 1)
        def _():
            pipeline(CPW1, NS * CPW0 + s * CPW1)

        plsc.subcore_barrier()
        pltpu.sync_copy(
            acc_sh.at[pl.ds(s * RPS, RPS)],
            out_hbm.at[pl.ds(c * N_PAD + s * RPS, RPS)],
        )

    return seg


@functools.partial(
    pl.kernel,
    out_type=jax.ShapeDtypeStruct((NC * N_PAD, 128), jnp.float32),
    mesh=_mesh,
    scratch_types=[
        pltpu.VMEM((CPW, CHUNK), jnp.int32),
        pltpu.VMEM((CHUNK, 128), jnp.float32),
        pltpu.VMEM_SHARED((N_PAD, 128), jnp.float32),
    ],
)
def _deg(dst_hbm, ones_hbm, zero_hbm, out_hbm, dst_v, ones_v, acc_sh):
    c = lax.axis_index("c")
    s = lax.axis_index("s")
    w = s * NC + c
    pltpu.sync_copy(zero_hbm, acc_sh.at[pl.ds(s * RPS, RPS)])
    pltpu.sync_copy(ones_hbm, ones_v)
    pltpu.sync_copy(dst_hbm.at[pl.ds(w * CPW, CPW)], dst_v)
    plsc.subcore_barrier()

    @pl.loop(0, CPW)
    def _(j):
        pltpu.sync_copy(ones_v, acc_sh.at[dst_v.at[j]], add=True)

    plsc.subcore_barrier()
    pltpu.sync_copy(
        acc_sh.at[pl.ds(s * RPS, RPS)],
        out_hbm.at[pl.ds(c * N_PAD + s * RPS, RPS)],
    )


def _mm(x, w):
    d_in, d_out = w.shape

    def body(x_ref, w_ref, o_ref):
        o_ref[...] = jnp.dot(x_ref[...], w_ref[...],
                             preferred_element_type=jnp.float32)

    return pl.pallas_call(
        body,
        grid=(N_PAD // BLK,),
        in_specs=[
            pl.BlockSpec((BLK, d_in), lambda i: (i, 0)),
            pl.BlockSpec((d_in, d_out), lambda i: (0, 0)),
        ],
        out_specs=pl.BlockSpec((BLK, d_out), lambda i: (i, 0)),
        out_shape=jax.ShapeDtypeStruct((N_PAD, d_out), jnp.float32),
    )(x, w)


def _normk(deg2, t0):
    def body(d0_ref, d1_ref, t_ref, tn_ref, n_ref):
        deg = d0_ref[..., 0] + d1_ref[..., 0]
        nrm = jnp.where(deg > 0, lax.rsqrt(jnp.maximum(deg, 1.0)), 0.0)
        tn_ref[...] = t_ref[...] * nrm[:, None]
        n_ref[...] = nrm

    nb = N_PAD // BLK
    return pl.pallas_call(
        body,
        grid=(nb,),
        in_specs=[
            pl.BlockSpec((BLK, 128), lambda i: (i, 0)),
            pl.BlockSpec((BLK, 128), lambda i, _nb=nb: (i + _nb, 0)),
            pl.BlockSpec((BLK, D_H), lambda i: (i, 0)),
        ],
        out_specs=[
            pl.BlockSpec((BLK, D_H), lambda i: (i, 0)),
            pl.BlockSpec((BLK,), lambda i: (i,)),
        ],
        out_shape=[
            jax.ShapeDtypeStruct((N_PAD, D_H), jnp.float32),
            jax.ShapeDtypeStruct((N_PAD,), jnp.float32),
        ],
    )(deg2, deg2, t0)


def _layerk(acc, nrm, b, w):
    d_in, d_out = w.shape

    def body(a0_ref, a1_ref, n_ref, b_ref, w_ref, o_ref):
        nv = n_ref[...][:, None]
        h = (a0_ref[...] + a1_ref[...]) * nv + b_ref[...][None, :]
        h = jnp.maximum(h, 0.0) * nv
        o_ref[...] = jnp.dot(h, w_ref[...], preferred_element_type=jnp.float32)

    nb = N_PAD // BLK
    return pl.pallas_call(
        body,
        grid=(nb,),
        in_specs=[
            pl.BlockSpec((BLK, d_in), lambda i: (i, 0)),
            pl.BlockSpec((BLK, d_in), lambda i, _nb=nb: (i + _nb, 0)),
            pl.BlockSpec((BLK,), lambda i: (i,)),
            pl.BlockSpec((d_in,), lambda i: (0,)),
            pl.BlockSpec((d_in, d_out), lambda i: (0, 0)),
        ],
        out_specs=pl.BlockSpec((BLK, d_out), lambda i: (i, 0)),
        out_shape=jax.ShapeDtypeStruct((N_PAD, d_out), jnp.float32),
    )(acc, acc, nrm, b, w)


def _finalk(acc, nrm, b):
    def body(a0_ref, a1_ref, n_ref, b_ref, o_ref):
        o_ref[...] = ((a0_ref[...] + a1_ref[...]) * n_ref[...][:, None]
                      + b_ref[...][None, :])

    nb = N_PAD // BLK
    return pl.pallas_call(
        body,
        grid=(nb,),
        in_specs=[
            pl.BlockSpec((BLK, C_PAD), lambda i: (i, 0)),
            pl.BlockSpec((BLK, C_PAD), lambda i, _nb=nb: (i + _nb, 0)),
            pl.BlockSpec((BLK,), lambda i: (i,)),
            pl.BlockSpec((C_PAD,), lambda i: (0,)),
        ],
        out_specs=pl.BlockSpec((BLK, C_PAD), lambda i: (i, 0)),
        out_shape=jax.ShapeDtypeStruct((N_PAD, C_PAD), jnp.float32),
    )(acc, acc, nrm, b)


_seg128 = _make_seg(D_H)
_seg64 = _make_seg(C_PAD)


def kernel(features, edge_index, W0, b0, W1, b1, W2, b2):
    f32 = jnp.float32
    pad_e = E_PAD - E
    src = jnp.concatenate(
        [edge_index[0], jnp.zeros((pad_e,), jnp.int32)]).reshape(NCHUNKS, CHUNK)
    dst = jnp.concatenate(
        [edge_index[1], jnp.full((pad_e,), N, jnp.int32)]).reshape(NCHUNKS, CHUNK)
    xp = jnp.concatenate(
        [features, jnp.zeros((N_PAD - N, D_IN), f32)], axis=0)
    zeros640 = jnp.zeros((RPS, 128), f32)
    ones128 = jnp.ones((CHUNK, 128), f32)
    W2p = jnp.concatenate(
        [W2, jnp.zeros((D_H, C_PAD - N_CLASSES), f32)], axis=1)
    b2p = jnp.concatenate([b2, jnp.zeros((C_PAD - N_CLASSES,), f32)])

    deg2 = _deg(dst, ones128, zeros640)
    t0 = _mm(xp, W0)
    t0n, nrm = _normk(deg2, t0)
    acc1 = _seg128(t0n, src, dst, zeros640)
    t1 = _layerk(acc1, nrm, b0, W1)
    acc2 = _seg128(t1, src, dst, zeros640)
    t2 = _layerk(acc2, nrm, b1, W2p)
    acc3 = _seg64(t2, src, dst, zeros640[:, :C_PAD])
    outp = _finalk(acc3, nrm, b2p)
    return outp[:N, :N_CLASSES]


# speedup vs baseline: 1.1545x; 1.1545x over previous
import functools

import jax
import jax.numpy as jnp
from jax import lax
from jax.experimental import pallas as pl
from jax.experimental.pallas import tpu as pltpu
from jax.experimental.pallas import tpu_sc as plsc

N = 10000
E = 320000
D_IN = 128
D_H = 128
N_CLASSES = 40
C_PAD = 128
N_PAD = 10240
NC, NS = 2, 16
NW = NC * NS
CHUNK = 128
NCHUNKS = 2560
E_PAD = NCHUNKS * CHUNK
CPW = NCHUNKS // NW
RPS = N_PAD // NS
SBLK = 40
CPW0, CPW1 = 120, 40
BLK = 256

_mesh = plsc.VectorSubcoreMesh(core_axis_name="c", subcore_axis_name="s")


def _make_seg(d):
    @functools.partial(
        pl.kernel,
        out_type=jax.ShapeDtypeStruct((NC * N_PAD, d), jnp.float32),
        mesh=_mesh,
        scratch_types=[
            pltpu.VMEM((SBLK, CHUNK), jnp.int32),
            pltpu.VMEM((SBLK, CHUNK), jnp.int32),
            pltpu.VMEM((CHUNK, d), jnp.float32),
            pltpu.VMEM((CHUNK, d), jnp.float32),
            pltpu.VMEM_SHARED((N_PAD, d), jnp.float32),
            pltpu.SemaphoreType.DMA,
            pltpu.SemaphoreType.DMA,
        ],
    )
    def seg(t_hbm, src_hbm, dst_hbm, zero_hbm, out_hbm, src_v, dst_v,
            rows0, rows1, acc_sh, ss0, ss1):
        c = lax.axis_index("c")
        s = lax.axis_index("s")
        pltpu.sync_copy(zero_hbm, acc_sh.at[pl.ds(s * RPS, RPS)])
        plsc.subcore_barrier()

        bufs = (rows0, rows1)
        sems = (ss0, ss1)

        def pipeline(cpw, base_w):
            for h in range(cpw // SBLK):
                base = base_w + h * SBLK
                pltpu.sync_copy(src_hbm.at[pl.ds(base, SBLK)], src_v)
                pltpu.sync_copy(dst_hbm.at[pl.ds(base, SBLK)], dst_v)
                for p in (0, 1):
                    pltpu.sync_copy(t_hbm.at[src_v.at[p]], bufs[p])
                    pltpu.async_copy(bufs[p], acc_sh.at[dst_v.at[p]],
                                     sems[p], add=True)

                @pl.loop(2, SBLK, step=2)
                def _(j):
                    for p in (0, 1):
                        jj = j + p
                        pltpu.make_async_copy(t_hbm.at[src_v.at[jj]], bufs[p],
                                              sems[p]).wait()
                        pltpu.sync_copy(t_hbm.at[src_v.at[jj]], bufs[p])
                        pltpu.async_copy(bufs[p], acc_sh.at[dst_v.at[jj]],
                                         sems[p], add=True)

                for p in (0, 1):
                    pltpu.make_async_copy(t_hbm.at[src_v.at[p]], bufs[p],
                                          sems[p]).wait()

        @pl.when(c == 0)
        def _():
            pipeline(CPW0, s * CPW0)

        @pl.when(c == 1)
        def _():
            pipeline(CPW1, NS * CPW0 + s * CPW1)

        plsc.subcore_barrier()
        pltpu.sync_copy(
            acc_sh.at[pl.ds(s * RPS, RPS)],
            out_hbm.at[pl.ds(c * N_PAD + s * RPS, RPS)],
        )

    return seg


@functools.partial(
    pl.kernel,
    out_type=jax.ShapeDtypeStruct((NC * N_PAD, 128), jnp.float32),
    mesh=_mesh,
    scratch_types=[
        pltpu.VMEM((CPW, CHUNK), jnp.int32),
        pltpu.VMEM((CHUNK, 128), jnp.float32),
        pltpu.VMEM_SHARED((N_PAD, 128), jnp.float32),
    ],
)
def _deg(dst_hbm, ones_hbm, zero_hbm, out_hbm, dst_v, ones_v, acc_sh):
    c = lax.axis_index("c")
    s = lax.axis_index("s")
    w = s * NC + c
    pltpu.sync_copy(zero_hbm, acc_sh.at[pl.ds(s * RPS, RPS)])
    pltpu.sync_copy(ones_hbm, ones_v)
    pltpu.sync_copy(dst_hbm.at[pl.ds(w * CPW, CPW)], dst_v)
    plsc.subcore_barrier()

    @pl.loop(0, CPW)
    def _(j):
        pltpu.sync_copy(ones_v, acc_sh.at[dst_v.at[j]], add=True)

    plsc.subcore_barrier()
    pltpu.sync_copy(
        acc_sh.at[pl.ds(s * RPS, RPS)],
        out_hbm.at[pl.ds(c * N_PAD + s * RPS, RPS)],
    )


def _mm(x, w):
    d_in, d_out = w.shape

    def body(x_ref, w_ref, o_ref):
        o_ref[...] = jnp.dot(x_ref[...], w_ref[...],
                             preferred_element_type=jnp.float32)

    return pl.pallas_call(
        body,
        grid=(N_PAD // BLK,),
        in_specs=[
            pl.BlockSpec((BLK, d_in), lambda i: (i, 0)),
            pl.BlockSpec((d_in, d_out), lambda i: (0, 0)),
        ],
        out_specs=pl.BlockSpec((BLK, d_out), lambda i: (i, 0)),
        out_shape=jax.ShapeDtypeStruct((N_PAD, d_out), jnp.float32),
    )(x, w)


def _normk(deg2, t0):
    def body(d0_ref, d1_ref, t_ref, tn_ref, n_ref):
        deg = d0_ref[..., 0] + d1_ref[..., 0]
        nrm = jnp.where(deg > 0, lax.rsqrt(jnp.maximum(deg, 1.0)), 0.0)
        tn_ref[...] = t_ref[...] * nrm[:, None]
        n_ref[...] = nrm

    nb = N_PAD // BLK
    return pl.pallas_call(
        body,
        grid=(nb,),
        in_specs=[
            pl.BlockSpec((BLK, 128), lambda i: (i, 0)),
            pl.BlockSpec((BLK, 128), lambda i, _nb=nb: (i + _nb, 0)),
            pl.BlockSpec((BLK, D_H), lambda i: (i, 0)),
        ],
        out_specs=[
            pl.BlockSpec((BLK, D_H), lambda i: (i, 0)),
            pl.BlockSpec((BLK,), lambda i: (i,)),
        ],
        out_shape=[
            jax.ShapeDtypeStruct((N_PAD, D_H), jnp.float32),
            jax.ShapeDtypeStruct((N_PAD,), jnp.float32),
        ],
    )(deg2, deg2, t0)


def _layerk(acc, nrm, b, w):
    d_in, d_out = w.shape

    def body(a0_ref, a1_ref, n_ref, b_ref, w_ref, o_ref):
        nv = n_ref[...][:, None]
        h = (a0_ref[...] + a1_ref[...]) * nv + b_ref[...][None, :]
        h = jnp.maximum(h, 0.0) * nv
        o_ref[...] = jnp.dot(h, w_ref[...], preferred_element_type=jnp.float32)

    nb = N_PAD // BLK
    return pl.pallas_call(
        body,
        grid=(nb,),
        in_specs=[
            pl.BlockSpec((BLK, d_in), lambda i: (i, 0)),
            pl.BlockSpec((BLK, d_in), lambda i, _nb=nb: (i + _nb, 0)),
            pl.BlockSpec((BLK,), lambda i: (i,)),
            pl.BlockSpec((d_in,), lambda i: (0,)),
            pl.BlockSpec((d_in, d_out), lambda i: (0, 0)),
        ],
        out_specs=pl.BlockSpec((BLK, d_out), lambda i: (i, 0)),
        out_shape=jax.ShapeDtypeStruct((N_PAD, d_out), jnp.float32),
    )(acc, acc, nrm, b, w)


def _finalk(acc, nrm, b):
    def body(a0_ref, a1_ref, n_ref, b_ref, o_ref):
        o_ref[...] = ((a0_ref[...] + a1_ref[...]) * n_ref[...][:, None]
                      + b_ref[...][None, :])

    nb = N_PAD // BLK
    return pl.pallas_call(
        body,
        grid=(nb,),
        in_specs=[
            pl.BlockSpec((BLK, C_PAD), lambda i: (i, 0)),
            pl.BlockSpec((BLK, C_PAD), lambda i, _nb=nb: (i + _nb, 0)),
            pl.BlockSpec((BLK,), lambda i: (i,)),
            pl.BlockSpec((C_PAD,), lambda i: (0,)),
        ],
        out_specs=pl.BlockSpec((BLK, C_PAD), lambda i: (i, 0)),
        out_shape=jax.ShapeDtypeStruct((N_PAD, C_PAD), jnp.float32),
    )(acc, acc, nrm, b)


_seg128 = _make_seg(D_H)
_seg64 = _make_seg(C_PAD)


def kernel(features, edge_index, W0, b0, W1, b1, W2, b2):
    f32 = jnp.float32
    pad_e = E_PAD - E
    src = jnp.concatenate(
        [edge_index[0], jnp.zeros((pad_e,), jnp.int32)]).reshape(NCHUNKS, CHUNK)
    dst = jnp.concatenate(
        [edge_index[1], jnp.full((pad_e,), N, jnp.int32)]).reshape(NCHUNKS, CHUNK)
    xp = jnp.concatenate(
        [features, jnp.zeros((N_PAD - N, D_IN), f32)], axis=0)
    zeros640 = jnp.zeros((RPS, 128), f32)
    ones128 = jnp.ones((CHUNK, 128), f32)
    W2p = jnp.concatenate(
        [W2, jnp.zeros((D_H, C_PAD - N_CLASSES), f32)], axis=1)
    b2p = jnp.concatenate([b2, jnp.zeros((C_PAD - N_CLASSES,), f32)])

    deg2 = _deg(dst, ones128, zeros640)
    t0 = _mm(xp, W0)
    t0n, nrm = _normk(deg2, t0)
    acc1 = _seg128(t0n, src, dst, zeros640)
    t1 = _layerk(acc1, nrm, b0, W1)
    acc2 = _seg128(t1, src, dst, zeros640)
    t2 = _layerk(acc2, nrm, b1, W2p)
    acc3 = _seg64(t2, src, dst, zeros640[:, :C_PAD])
    outp = _finalk(acc3, nrm, b2p)
    return outp[:N, :N_CLASSES]
